# Initial kernel scaffold; baseline (speedup 1.0000x reference)
import jax
import jax.numpy as jnp
from jax.experimental import pallas as pl


def kernel(x_host, x_flow, ei_src_of, ei_dst_of, ei_rev_src_of, ei_rev_dst_of, W_r1, att_s_r1, att_d_r1, b_r1, W_r2, att_s_r2, att_d_r2, b_r2, W_r3, att_s_r3, att_d_r3, b_r3, W_r4, att_s_r4, att_d_r4, b_r4, W_lin, b_lin):
    raise NotImplementedError("write your pallas kernel here")


# speedup vs baseline: 24.8229x; 24.8229x over previous
import functools

import jax
import jax.numpy as jnp
from jax import lax
from jax.experimental import pallas as pl
from jax.experimental.pallas import tpu as pltpu
from jax.experimental.pallas import tpu_sc as plsc

N_HOST = 10000
N_FLOW = 10000
E_EDGES = 320000
D_IN = 128
H_FEAT = 128
N_OUT = 2

TILES = 16
CW = 80
NB = 10
BLOCKS = 25
CHUNKS = NB * BLOCKS
EPT = CHUNKS * CW
NPAD = 10112
ROWS_PT = NPAD // TILES
NEG_SLOPE = 0.2

_BLK = 2000
_GRID = N_HOST // _BLK


def _prep_body(xh_ref, xf_ref, w1_ref, w2_ref, as1_ref, ad1_ref, as2_ref,
               ad2_ref, h_ref, a_ref):
    xh = xh_ref[...]
    xf = xf_ref[...]
    rows = []
    avec = []
    for w_ref, as_ref, ad_ref in ((w1_ref, as1_ref, ad1_ref),
                                  (w2_ref, as2_ref, ad2_ref)):
        w = w_ref[...]
        h = jnp.dot(xh, w, preferred_element_type=jnp.float32)
        hd = jnp.dot(xf, w, preferred_element_type=jnp.float32)
        rows.append(h)
        avec.append(jnp.sum(h * as_ref[...], axis=1))
        avec.append(jnp.sum(hd * ad_ref[...], axis=1))
    h_ref[...] = jnp.stack(rows)
    a_ref[...] = jnp.stack(avec, axis=1)


def _prep(x_host, x_flow, w1, w2, as1, ad1, as2, ad2):
    full = lambda shape: pl.BlockSpec(shape, lambda i: (0,) * len(shape))
    return pl.pallas_call(
        _prep_body,
        grid=(_GRID,),
        in_specs=[
            pl.BlockSpec((_BLK, D_IN), lambda i: (i, 0)),
            pl.BlockSpec((_BLK, D_IN), lambda i: (i, 0)),
            full((D_IN, H_FEAT)),
            full((D_IN, H_FEAT)),
            full((1, H_FEAT)),
            full((1, H_FEAT)),
            full((1, H_FEAT)),
            full((1, H_FEAT)),
        ],
        out_specs=[
            pl.BlockSpec((2, _BLK, H_FEAT), lambda i: (0, i, 0)),
            pl.BlockSpec((_BLK, 4), lambda i: (i, 0)),
        ],
        out_shape=[
            jax.ShapeDtypeStruct((2, N_HOST, H_FEAT), jnp.float32),
            jax.ShapeDtypeStruct((N_HOST, 4), jnp.float32),
        ],
    )(x_host, x_flow, w1, w2, as1, ad1, as2, ad2)


def _sc_body(h_ref, a_ref, src_ref, dst_ref, z_ref, zd_ref,
             acc_out, den_out,
             srcbuf, dstbuf, a_s, a_d, exc, rows, den_l, acc, gsem):
    c = lax.axis_index("c")
    t = lax.axis_index("s")
    w = c * TILES + t

    pltpu.sync_copy(a_ref.at[pl.ds((2 * c) * N_HOST, N_HOST)], a_s)
    pltpu.sync_copy(a_ref.at[pl.ds((2 * c + 1) * N_HOST, N_HOST)], a_d)
    pltpu.sync_copy(z_ref, acc.at[pl.ds(t * ROWS_PT, ROWS_PT)])
    pltpu.sync_copy(zd_ref, den_l)
    plsc.subcore_barrier()

    src_t = src_ref.at[w]
    dst_t = dst_ref.at[w]
    a_base = c * N_HOST
    iota16 = lax.iota(jnp.int32, 16)

    @pl.loop(0, BLOCKS)
    def _(blk):
        pltpu.sync_copy(src_t.at[blk], srcbuf)
        pltpu.sync_copy(dst_t.at[blk], dstbuf)
        for jj in range(NB):
            for v in range(CW // 16):
                sv = srcbuf[jj, pl.ds(v * 16, 16)] - a_base
                dv = dstbuf[jj, pl.ds(v * 16, 16)]
                e = plsc.load_gather(a_s, [sv]) + plsc.load_gather(a_d, [dv])
                e = jnp.where(e >= 0.0, e, NEG_SLOPE * e)
                ex = jnp.exp(e)
                exc[pl.ds(v * 16, 16)] = ex

                k, vv = plsc.sort_key_val(dv, ex)
                for s in (1, 2, 4, 8):
                    prev = jnp.maximum(iota16 - s, 0)
                    kprev = k.at[prev].get(mode="promise_in_bounds")
                    vprev = vv.at[prev].get(mode="promise_in_bounds")
                    same = (iota16 >= s) & (k == kprev)
                    vv = vv + jnp.where(same, vprev, 0.0)
                nxt = k.at[jnp.minimum(iota16 + 1, 15)].get(
                    mode="promise_in_bounds")
                last = (iota16 == 15) | (k != nxt)
                plsc.addupdate_scatter(den_l, [k], vv, mask=last)

            pltpu.async_copy(h_ref.at[srcbuf.at[jj]], rows, gsem).wait()

            @pl.loop(0, CW)
            def _(rw):
                ev = plsc.load_gather(exc, [jnp.full((16,), rw, jnp.int32)])
                for k9 in range(H_FEAT // 16):
                    sl = rows[rw, pl.ds(k9 * 16, 16)]
                    rows[rw, pl.ds(k9 * 16, 16)] = sl * ev

            pltpu.sync_copy(rows, acc.at[dstbuf.at[jj]], add=True)

    plsc.subcore_barrier()
    pltpu.sync_copy(acc.at[pl.ds(t * ROWS_PT, ROWS_PT)],
                    acc_out.at[pl.ds(c * NPAD + t * ROWS_PT, ROWS_PT)])
    pltpu.sync_copy(den_l, den_out.at[pl.ds(w * NPAD, NPAD)])


def _sc_gat(h_tbl, a_flat, src_all, dst_all, zeros, zeros_d):
    mesh = plsc.VectorSubcoreMesh(core_axis_name="c", subcore_axis_name="s")
    f = pl.kernel(
        _sc_body,
        out_type=(
            jax.ShapeDtypeStruct((2 * NPAD, H_FEAT), jnp.float32),
            jax.ShapeDtypeStruct((2 * TILES * NPAD,), jnp.float32),
        ),
        mesh=mesh,
        compiler_params=pltpu.CompilerParams(needs_layout_passes=False),
        scratch_types=[
            pltpu.VMEM((NB, CW), jnp.int32),
            pltpu.VMEM((NB, CW), jnp.int32),
            pltpu.VMEM((N_HOST,), jnp.float32),
            pltpu.VMEM((N_FLOW,), jnp.float32),
            pltpu.VMEM((CW,), jnp.float32),
            pltpu.VMEM((CW, H_FEAT), jnp.float32),
            pltpu.VMEM((NPAD,), jnp.float32),
            pltpu.VMEM_SHARED((NPAD, H_FEAT), jnp.float32),
            pltpu.SemaphoreType.DMA,
        ],
    )
    return f(h_tbl, a_flat, src_all, dst_all, zeros, zeros_d)


def _final_body(acc_ref, den_ref, b12_ref, wl_ref, bl_ref, out_ref):
    a = acc_ref[...]
    d = jnp.sum(den_ref[...], axis=2)
    d = jnp.maximum(d, 1e-16)
    flow = a[0] / d[0][:, None] + a[1] / d[1][:, None] + b12_ref[...]
    flow = jnp.maximum(flow, 0.0)
    out_ref[...] = (
        jnp.dot(flow, wl_ref[...], preferred_element_type=jnp.float32)
        + bl_ref[...])


def _final(acc, dens, b12, w_lin, b_lin):
    full = lambda shape: pl.BlockSpec(shape, lambda i: (0,) * len(shape))
    return pl.pallas_call(
        _final_body,
        grid=(_GRID,),
        in_specs=[
            pl.BlockSpec((2, _BLK, H_FEAT), lambda i: (0, i, 0)),
            pl.BlockSpec((2, _BLK, TILES), lambda i: (0, i, 0)),
            full((1, H_FEAT)),
            full((H_FEAT, N_OUT)),
            full((1, N_OUT)),
        ],
        out_specs=pl.BlockSpec((_BLK, N_OUT), lambda i: (i, 0)),
        out_shape=jax.ShapeDtypeStruct((N_FLOW, N_OUT), jnp.float32),
    )(acc, dens, b12, w_lin, b_lin)


def kernel(x_host, x_flow, ei_src_of, ei_dst_of, ei_rev_src_of, ei_rev_dst_of,
           W_r1, att_s_r1, att_d_r1, b_r1, W_r2, att_s_r2, att_d_r2, b_r2,
           W_r3, att_s_r3, att_d_r3, b_r3, W_r4, att_s_r4, att_d_r4, b_r4,
           W_lin, b_lin):
    h_tbl, a_tbl = _prep(
        x_host, x_flow, W_r1, W_r2,
        att_s_r1.reshape(1, H_FEAT), att_d_r1.reshape(1, H_FEAT),
        att_s_r2.reshape(1, H_FEAT), att_d_r2.reshape(1, H_FEAT))

    h_tbl = h_tbl.reshape(2 * N_HOST, H_FEAT)
    a_flat = a_tbl.T.reshape(4 * N_HOST)
    src_all = jnp.stack([ei_src_of[0], ei_dst_of[0] + N_HOST])
    dst_all = jnp.stack([ei_src_of[1], ei_dst_of[1]])
    src_all = src_all.astype(jnp.int32).reshape(2 * TILES, BLOCKS, NB, CW)
    dst_all = dst_all.astype(jnp.int32).reshape(2 * TILES, BLOCKS, NB, CW)
    zeros = jnp.zeros((ROWS_PT, H_FEAT), jnp.float32)
    zeros_d = jnp.zeros((NPAD,), jnp.float32)

    acc, dens = _sc_gat(h_tbl, a_flat, src_all, dst_all, zeros, zeros_d)
    acc = acc.reshape(2, NPAD, H_FEAT)[:, :N_FLOW, :]
    dens = dens.reshape(2, TILES, NPAD)[:, :, :N_FLOW]
    dens = dens.transpose(0, 2, 1)

    b12 = (b_r1 + b_r2).reshape(1, H_FEAT)
    return _final(acc, dens, b12, W_lin, b_lin.reshape(1, N_OUT))


# speedup vs baseline: 25.8407x; 1.0410x over previous
import functools

import jax
import jax.numpy as jnp
from jax import lax
from jax.experimental import pallas as pl
from jax.experimental.pallas import tpu as pltpu
from jax.experimental.pallas import tpu_sc as plsc

N_HOST = 10000
N_FLOW = 10000
E_EDGES = 320000
D_IN = 128
H_FEAT = 128
N_OUT = 2

TILES = 16
CW = 80
NB = 10
BLOCKS = 25
CHUNKS = NB * BLOCKS
EPT = CHUNKS * CW
NPAD = 10112
ROWS_PT = NPAD // TILES
NEG_SLOPE = 0.2

_BLK = 2000
_GRID = N_HOST // _BLK


def _prep_body(xh_ref, xf_ref, w1_ref, w2_ref, as1_ref, ad1_ref, as2_ref,
               ad2_ref, h_ref, a_ref):
    xh = xh_ref[...]
    xf = xf_ref[...]
    rows = []
    avec = []
    for w_ref, as_ref, ad_ref in ((w1_ref, as1_ref, ad1_ref),
                                  (w2_ref, as2_ref, ad2_ref)):
        w = w_ref[...]
        h = jnp.dot(xh, w, preferred_element_type=jnp.float32)
        hd = jnp.dot(xf, w, preferred_element_type=jnp.float32)
        rows.append(h)
        avec.append(jnp.sum(h * as_ref[...], axis=1))
        avec.append(jnp.sum(hd * ad_ref[...], axis=1))
    h_ref[...] = jnp.stack(rows)
    a_ref[...] = jnp.stack(avec, axis=1)


def _prep(x_host, x_flow, w1, w2, as1, ad1, as2, ad2):
    full = lambda shape: pl.BlockSpec(shape, lambda i: (0,) * len(shape))
    return pl.pallas_call(
        _prep_body,
        grid=(_GRID,),
        in_specs=[
            pl.BlockSpec((_BLK, D_IN), lambda i: (i, 0)),
            pl.BlockSpec((_BLK, D_IN), lambda i: (i, 0)),
            full((D_IN, H_FEAT)),
            full((D_IN, H_FEAT)),
            full((1, H_FEAT)),
            full((1, H_FEAT)),
            full((1, H_FEAT)),
            full((1, H_FEAT)),
        ],
        out_specs=[
            pl.BlockSpec((2, _BLK, H_FEAT), lambda i: (0, i, 0)),
            pl.BlockSpec((_BLK, 4), lambda i: (i, 0)),
        ],
        out_shape=[
            jax.ShapeDtypeStruct((2, N_HOST, H_FEAT), jnp.float32),
            jax.ShapeDtypeStruct((N_HOST, 4), jnp.float32),
        ],
    )(x_host, x_flow, w1, w2, as1, ad1, as2, ad2)


def _sc_coef_body(a_ref, src_ref, dst_ref, zd_ref,
                  ex_out, den_out,
                  srcbuf, dstbuf, a_s, a_d, exc, den_l):
    c = lax.axis_index("c")
    t = lax.axis_index("s")
    w = c * TILES + t

    pltpu.sync_copy(a_ref.at[pl.ds((2 * c) * N_HOST, N_HOST)], a_s)
    pltpu.sync_copy(a_ref.at[pl.ds((2 * c + 1) * N_HOST, N_HOST)], a_d)
    pltpu.sync_copy(zd_ref, den_l)

    src_t = src_ref.at[w]
    dst_t = dst_ref.at[w]
    a_base = c * N_HOST
    iota16 = lax.iota(jnp.int32, 16)

    @pl.loop(0, BLOCKS)
    def _(blk):
        pltpu.sync_copy(src_t.at[blk], srcbuf)
        pltpu.sync_copy(dst_t.at[blk], dstbuf)
        for jj in range(NB):
            for v in range(CW // 16):
                sv = srcbuf[jj, pl.ds(v * 16, 16)] - a_base
                dv = dstbuf[jj, pl.ds(v * 16, 16)]
                e = plsc.load_gather(a_s, [sv]) + plsc.load_gather(a_d, [dv])
                e = jnp.where(e >= 0.0, e, NEG_SLOPE * e)
                ex = jnp.exp(e)
                exc[pl.ds(jj * CW + v * 16, 16)] = ex

                k, vv = plsc.sort_key_val(dv, ex)
                for s in (1, 2, 4, 8):
                    prev = jnp.maximum(iota16 - s, 0)
                    kprev = k.at[prev].get(mode="promise_in_bounds")
                    vprev = vv.at[prev].get(mode="promise_in_bounds")
                    same = (iota16 >= s) & (k == kprev)
                    vv = vv + jnp.where(same, vprev, 0.0)
                nxt = k.at[jnp.minimum(iota16 + 1, 15)].get(
                    mode="promise_in_bounds")
                last = (iota16 == 15) | (k != nxt)
                plsc.addupdate_scatter(den_l, [k], vv, mask=last)

        pltpu.sync_copy(
            exc, ex_out.at[pl.ds(w * EPT + blk * NB * CW, NB * CW)])

    pltpu.sync_copy(den_l, den_out.at[pl.ds(w * N_FLOW, N_FLOW)])


def _sc_coef(a_flat, src_all, dst_all, zeros_d):
    mesh = plsc.VectorSubcoreMesh(core_axis_name="c", subcore_axis_name="s")
    f = pl.kernel(
        _sc_coef_body,
        out_type=(
            jax.ShapeDtypeStruct((2 * TILES * EPT,), jnp.float32),
            jax.ShapeDtypeStruct((2 * TILES * N_FLOW,), jnp.float32),
        ),
        mesh=mesh,
        compiler_params=pltpu.CompilerParams(needs_layout_passes=False),
        scratch_types=[
            pltpu.VMEM((NB, CW), jnp.int32),
            pltpu.VMEM((NB, CW), jnp.int32),
            pltpu.VMEM((N_HOST,), jnp.float32),
            pltpu.VMEM((N_FLOW,), jnp.float32),
            pltpu.VMEM((NB * CW,), jnp.float32),
            pltpu.VMEM((N_FLOW,), jnp.float32),
        ],
    )
    return f(a_flat, src_all, dst_all, zeros_d)


def _sc_rows_body(h_ref, ex_ref, src_ref, dst_ref, z_ref,
                  acc_out,
                  srcbuf, dstbuf, exblk, rows0, rows1,
                  acc, gsem0, gsem1, ssem0, ssem1):
    c = lax.axis_index("c")
    t = lax.axis_index("s")
    w = c * TILES + t

    pltpu.sync_copy(z_ref, acc.at[pl.ds(t * ROWS_PT, ROWS_PT)])
    plsc.subcore_barrier()

    src_t = src_ref.at[w]
    dst_t = dst_ref.at[w]
    bufs = (rows0, rows1)
    gsems = (gsem0, gsem1)
    ssems = (ssem0, ssem1)

    @pl.loop(0, BLOCKS)
    def _(blk):
        @pl.when(blk >= 1)
        def _():
            pltpu.make_async_copy(
                bufs[0], acc.at[dstbuf.at[NB - 2]], ssems[0]).wait()
            pltpu.make_async_copy(
                bufs[1], acc.at[dstbuf.at[NB - 1]], ssems[1]).wait()

        pltpu.sync_copy(src_t.at[blk], srcbuf)
        pltpu.sync_copy(dst_t.at[blk], dstbuf)
        pltpu.sync_copy(
            ex_ref.at[pl.ds(w * EPT + blk * NB * CW, NB * CW)], exblk)
        for jj in range(NB):
            b = jj % 2

            if jj >= 2:
                pltpu.make_async_copy(
                    bufs[b], acc.at[dstbuf.at[jj - 2]], ssems[b]).wait()

            pltpu.async_copy(h_ref.at[srcbuf.at[jj]], bufs[b], gsems[b]).wait()

            @pl.loop(0, CW)
            def _(rw):
                ev = plsc.load_gather(
                    exblk, [jnp.full((16,), jj * CW + rw, jnp.int32)])
                for k9 in range(H_FEAT // 16):
                    sl = bufs[b][rw, pl.ds(k9 * 16, 16)]
                    bufs[b][rw, pl.ds(k9 * 16, 16)] = sl * ev

            pltpu.async_copy(bufs[b], acc.at[dstbuf.at[jj]], ssems[b],
                             add=True)

    pltpu.make_async_copy(bufs[0], acc.at[dstbuf.at[NB - 2]], ssems[0]).wait()
    pltpu.make_async_copy(bufs[1], acc.at[dstbuf.at[NB - 1]], ssems[1]).wait()
    plsc.subcore_barrier()
    pltpu.sync_copy(acc.at[pl.ds(t * ROWS_PT, ROWS_PT)],
                    acc_out.at[pl.ds(c * NPAD + t * ROWS_PT, ROWS_PT)])


def _sc_rows(h_tbl, ex_flat, src_all, dst_all, zeros):
    mesh = plsc.VectorSubcoreMesh(core_axis_name="c", subcore_axis_name="s")
    f = pl.kernel(
        _sc_rows_body,
        out_type=jax.ShapeDtypeStruct((2 * NPAD, H_FEAT), jnp.float32),
        mesh=mesh,
        compiler_params=pltpu.CompilerParams(needs_layout_passes=False),
        scratch_types=[
            pltpu.VMEM((NB, CW), jnp.int32),
            pltpu.VMEM((NB, CW), jnp.int32),
            pltpu.VMEM((NB * CW,), jnp.float32),
            pltpu.VMEM((CW, H_FEAT), jnp.float32),
            pltpu.VMEM((CW, H_FEAT), jnp.float32),
            pltpu.VMEM_SHARED((NPAD, H_FEAT), jnp.float32),
            pltpu.SemaphoreType.DMA,
            pltpu.SemaphoreType.DMA,
            pltpu.SemaphoreType.DMA,
            pltpu.SemaphoreType.DMA,
        ],
    )
    return f(h_tbl, ex_flat, src_all, dst_all, zeros)


def _final_body(acc_ref, den_ref, b12_ref, wl_ref, bl_ref, out_ref):
    a = acc_ref[...]
    d = jnp.sum(den_ref[...], axis=2)
    d = jnp.maximum(d, 1e-16)
    flow = a[0] / d[0][:, None] + a[1] / d[1][:, None] + b12_ref[...]
    flow = jnp.maximum(flow, 0.0)
    out_ref[...] = (
        jnp.dot(flow, wl_ref[...], preferred_element_type=jnp.float32)
        + bl_ref[...])


def _final(acc, dens, b12, w_lin, b_lin):
    full = lambda shape: pl.BlockSpec(shape, lambda i: (0,) * len(shape))
    return pl.pallas_call(
        _final_body,
        grid=(_GRID,),
        in_specs=[
            pl.BlockSpec((2, _BLK, H_FEAT), lambda i: (0, i, 0)),
            pl.BlockSpec((2, _BLK, TILES), lambda i: (0, i, 0)),
            full((1, H_FEAT)),
            full((H_FEAT, N_OUT)),
            full((1, N_OUT)),
        ],
        out_specs=pl.BlockSpec((_BLK, N_OUT), lambda i: (i, 0)),
        out_shape=jax.ShapeDtypeStruct((N_FLOW, N_OUT), jnp.float32),
    )(acc, dens, b12, w_lin, b_lin)


def kernel(x_host, x_flow, ei_src_of, ei_dst_of, ei_rev_src_of, ei_rev_dst_of,
           W_r1, att_s_r1, att_d_r1, b_r1, W_r2, att_s_r2, att_d_r2, b_r2,
           W_r3, att_s_r3, att_d_r3, b_r3, W_r4, att_s_r4, att_d_r4, b_r4,
           W_lin, b_lin):
    h_tbl, a_tbl = _prep(
        x_host, x_flow, W_r1, W_r2,
        att_s_r1.reshape(1, H_FEAT), att_d_r1.reshape(1, H_FEAT),
        att_s_r2.reshape(1, H_FEAT), att_d_r2.reshape(1, H_FEAT))

    h_tbl = h_tbl.reshape(2 * N_HOST, H_FEAT)
    a_flat = a_tbl.T.reshape(4 * N_HOST)
    src_all = jnp.stack([ei_src_of[0], ei_dst_of[0] + N_HOST])
    dst_all = jnp.stack([ei_src_of[1], ei_dst_of[1]])
    src_all = src_all.astype(jnp.int32).reshape(2 * TILES, BLOCKS, NB, CW)
    dst_all = dst_all.astype(jnp.int32).reshape(2 * TILES, BLOCKS, NB, CW)
    zeros = jnp.zeros((ROWS_PT, H_FEAT), jnp.float32)
    zeros_d = jnp.zeros((N_FLOW,), jnp.float32)

    ex_flat, dens = _sc_coef(a_flat, src_all, dst_all, zeros_d)
    acc = _sc_rows(h_tbl, ex_flat, src_all, dst_all, zeros)
    acc = acc.reshape(2, NPAD, H_FEAT)[:, :N_FLOW, :]
    dens = dens.reshape(2, TILES, N_FLOW)
    dens = dens.transpose(0, 2, 1)

    b12 = (b_r1 + b_r2).reshape(1, H_FEAT)
    return _final(acc, dens, b12, W_lin, b_lin.reshape(1, N_OUT))


# speedup vs baseline: 37.1599x; 1.4380x over previous
import functools

import jax
import jax.numpy as jnp
from jax import lax
from jax.experimental import pallas as pl
from jax.experimental.pallas import tpu as pltpu
from jax.experimental.pallas import tpu_sc as plsc

N_HOST = 10000
N_FLOW = 10000
E_EDGES = 320000
D_IN = 128
H_FEAT = 128
N_OUT = 2

TILES = 16
CW = 80
NB = 10
BLOCKS = 25
CHUNKS = NB * BLOCKS
EPT = CHUNKS * CW
NPAD = 10112
ROWS_PT = NPAD // TILES
NEG_SLOPE = 0.2

_BLK = 2000
_GRID = N_HOST // _BLK


def _prep_body(xh_ref, xf_ref, w1_ref, w2_ref, as1_ref, ad1_ref, as2_ref,
               ad2_ref, h_ref, a_ref):
    xh = xh_ref[...]
    xf = xf_ref[...]
    rows = []
    avec = []
    for w_ref, as_ref, ad_ref in ((w1_ref, as1_ref, ad1_ref),
                                  (w2_ref, as2_ref, ad2_ref)):
        w = w_ref[...]
        h = jnp.dot(xh, w, preferred_element_type=jnp.float32)
        hd = jnp.dot(xf, w, preferred_element_type=jnp.float32)
        rows.append(h)
        avec.append(jnp.sum(h * as_ref[...], axis=1))
        avec.append(jnp.sum(hd * ad_ref[...], axis=1))
    h_ref[...] = jnp.stack(rows)
    a_ref[...] = jnp.stack(avec, axis=1)


def _prep(x_host, x_flow, w1, w2, as1, ad1, as2, ad2):
    full = lambda shape: pl.BlockSpec(shape, lambda i: (0,) * len(shape))
    return pl.pallas_call(
        _prep_body,
        grid=(_GRID,),
        in_specs=[
            pl.BlockSpec((_BLK, D_IN), lambda i: (i, 0)),
            pl.BlockSpec((_BLK, D_IN), lambda i: (i, 0)),
            full((D_IN, H_FEAT)),
            full((D_IN, H_FEAT)),
            full((1, H_FEAT)),
            full((1, H_FEAT)),
            full((1, H_FEAT)),
            full((1, H_FEAT)),
        ],
        out_specs=[
            pl.BlockSpec((2, _BLK, H_FEAT), lambda i: (0, i, 0)),
            pl.BlockSpec((_BLK, 4), lambda i: (i, 0)),
        ],
        out_shape=[
            jax.ShapeDtypeStruct((2, N_HOST, H_FEAT), jnp.float32),
            jax.ShapeDtypeStruct((N_HOST, 4), jnp.float32),
        ],
    )(x_host, x_flow, w1, w2, as1, ad1, as2, ad2)


def _sc_coef_body(a_ref, src_ref, dst_ref, zd_ref,
                  ex_out, den_out,
                  srcbuf, dstbuf, a_s, a_d, exc, den_l):
    c = lax.axis_index("c")
    t = lax.axis_index("s")
    w = c * TILES + t

    pltpu.sync_copy(a_ref.at[pl.ds((2 * c) * N_HOST, N_HOST)], a_s)
    pltpu.sync_copy(a_ref.at[pl.ds((2 * c + 1) * N_HOST, N_HOST)], a_d)
    pltpu.sync_copy(zd_ref, den_l)

    src_t = src_ref.at[w]
    dst_t = dst_ref.at[w]
    a_base = c * N_HOST
    iota16 = lax.iota(jnp.int32, 16)

    @pl.loop(0, BLOCKS)
    def _(blk):
        pltpu.sync_copy(src_t.at[blk], srcbuf)
        pltpu.sync_copy(dst_t.at[blk], dstbuf)
        for jj in range(NB):
            for v in range(CW // 16):
                sv = srcbuf[jj, pl.ds(v * 16, 16)] - a_base
                dv = dstbuf[jj, pl.ds(v * 16, 16)]
                e = plsc.load_gather(a_s, [sv]) + plsc.load_gather(a_d, [dv])
                e = jnp.where(e >= 0.0, e, NEG_SLOPE * e)
                ex = jnp.exp(e)
                exc[pl.ds(jj * CW + v * 16, 16)] = ex

                k, vv = plsc.sort_key_val(dv, ex)
                for s in (1, 2, 4, 8):
                    prev = jnp.maximum(iota16 - s, 0)
                    kprev = k.at[prev].get(mode="promise_in_bounds")
                    vprev = vv.at[prev].get(mode="promise_in_bounds")
                    same = (iota16 >= s) & (k == kprev)
                    vv = vv + jnp.where(same, vprev, 0.0)
                nxt = k.at[jnp.minimum(iota16 + 1, 15)].get(
                    mode="promise_in_bounds")
                last = (iota16 == 15) | (k != nxt)
                plsc.addupdate_scatter(den_l, [k], vv, mask=last)

        pltpu.sync_copy(
            exc, ex_out.at[pl.ds(w * EPT + blk * NB * CW, NB * CW)])

    pltpu.sync_copy(den_l, den_out.at[pl.ds(w * N_FLOW, N_FLOW)])


def _sc_coef(a_flat, src_all, dst_all, zeros_d):
    mesh = plsc.VectorSubcoreMesh(core_axis_name="c", subcore_axis_name="s")
    f = pl.kernel(
        _sc_coef_body,
        out_type=(
            jax.ShapeDtypeStruct((2 * TILES * EPT,), jnp.float32),
            jax.ShapeDtypeStruct((2 * TILES * N_FLOW,), jnp.float32),
        ),
        mesh=mesh,
        compiler_params=pltpu.CompilerParams(needs_layout_passes=False),
        scratch_types=[
            pltpu.VMEM((NB, CW), jnp.int32),
            pltpu.VMEM((NB, CW), jnp.int32),
            pltpu.VMEM((N_HOST,), jnp.float32),
            pltpu.VMEM((N_FLOW,), jnp.float32),
            pltpu.VMEM((NB * CW,), jnp.float32),
            pltpu.VMEM((N_FLOW,), jnp.float32),
        ],
    )
    return f(a_flat, src_all, dst_all, zeros_d)


def _sc_rows_body(h_ref, ex_ref, src_ref, dst_ref, z_ref,
                  acc_out,
                  srcbuf, dstbuf, exblk, rows0, rows1, rows2, rows3,
                  acc, gsem0, gsem1, gsem2, gsem3,
                  ssem0, ssem1, ssem2, ssem3):
    c = lax.axis_index("c")
    t = lax.axis_index("s")
    w = c * TILES + t

    pltpu.sync_copy(z_ref, acc.at[pl.ds(t * ROWS_PT, ROWS_PT)])
    plsc.subcore_barrier()

    src_t = src_ref.at[w]
    dst_t = dst_ref.at[w]
    bufs = (rows0, rows1, rows2, rows3)
    gsems = (gsem0, gsem1, gsem2, gsem3)
    ssems = (ssem0, ssem1, ssem2, ssem3)

    @pl.loop(0, BLOCKS)
    def _(blk):
        @pl.when(blk >= 1)
        def _():
            for q in range(NB - 4, NB):
                pltpu.make_async_copy(
                    bufs[q % 4], acc.at[dstbuf.at[q]], ssems[q % 4]).wait()

        pltpu.sync_copy(src_t.at[blk], srcbuf)
        pltpu.sync_copy(dst_t.at[blk], dstbuf)
        pltpu.sync_copy(
            ex_ref.at[pl.ds(w * EPT + blk * NB * CW, NB * CW)], exblk)

        pltpu.async_copy(h_ref.at[srcbuf.at[0]], bufs[0], gsems[0])
        pltpu.async_copy(h_ref.at[srcbuf.at[1]], bufs[1], gsems[1])

        for jj in range(NB):
            b = jj % 4

            pltpu.make_async_copy(
                h_ref.at[srcbuf.at[jj]], bufs[b], gsems[b]).wait()

            if jj + 2 < NB:
                bn = (jj + 2) % 4
                if jj >= 2:
                    pltpu.make_async_copy(
                        bufs[bn], acc.at[dstbuf.at[jj - 2]],
                        ssems[bn]).wait()
                pltpu.async_copy(h_ref.at[srcbuf.at[jj + 2]], bufs[bn],
                                 gsems[bn])

            @pl.loop(0, CW)
            def _(rw):
                ev = plsc.load_gather(
                    exblk, [jnp.full((16,), jj * CW + rw, jnp.int32)])
                for k9 in range(H_FEAT // 16):
                    sl = bufs[b][rw, pl.ds(k9 * 16, 16)]
                    bufs[b][rw, pl.ds(k9 * 16, 16)] = sl * ev

            pltpu.async_copy(bufs[b], acc.at[dstbuf.at[jj]], ssems[b],
                             add=True)

    for q in range(NB - 4, NB):
        pltpu.make_async_copy(
            bufs[q % 4], acc.at[dstbuf.at[q]], ssems[q % 4]).wait()
    plsc.subcore_barrier()
    pltpu.sync_copy(acc.at[pl.ds(t * ROWS_PT, ROWS_PT)],
                    acc_out.at[pl.ds(c * NPAD + t * ROWS_PT, ROWS_PT)])


def _sc_rows(h_tbl, ex_flat, src_all, dst_all, zeros):
    mesh = plsc.VectorSubcoreMesh(core_axis_name="c", subcore_axis_name="s")
    f = pl.kernel(
        _sc_rows_body,
        out_type=jax.ShapeDtypeStruct((2 * NPAD, H_FEAT), jnp.float32),
        mesh=mesh,
        compiler_params=pltpu.CompilerParams(needs_layout_passes=False),
        scratch_types=[
            pltpu.VMEM((NB, CW), jnp.int32),
            pltpu.VMEM((NB, CW), jnp.int32),
            pltpu.VMEM((NB * CW,), jnp.float32),
            pltpu.VMEM((CW, H_FEAT), jnp.float32),
            pltpu.VMEM((CW, H_FEAT), jnp.float32),
            pltpu.VMEM((CW, H_FEAT), jnp.float32),
            pltpu.VMEM((CW, H_FEAT), jnp.float32),
            pltpu.VMEM_SHARED((NPAD, H_FEAT), jnp.float32),
            pltpu.SemaphoreType.DMA,
            pltpu.SemaphoreType.DMA,
            pltpu.SemaphoreType.DMA,
            pltpu.SemaphoreType.DMA,
            pltpu.SemaphoreType.DMA,
            pltpu.SemaphoreType.DMA,
            pltpu.SemaphoreType.DMA,
            pltpu.SemaphoreType.DMA,
        ],
    )
    return f(h_tbl, ex_flat, src_all, dst_all, zeros)


def _final_body(acc_ref, den_ref, b12_ref, wl_ref, bl_ref, out_ref):
    a = acc_ref[...]
    d = jnp.sum(den_ref[...], axis=2)
    d = jnp.maximum(d, 1e-16)
    flow = a[0] / d[0][:, None] + a[1] / d[1][:, None] + b12_ref[...]
    flow = jnp.maximum(flow, 0.0)
    out_ref[...] = (
        jnp.dot(flow, wl_ref[...], preferred_element_type=jnp.float32)
        + bl_ref[...])


def _final(acc, dens, b12, w_lin, b_lin):
    full = lambda shape: pl.BlockSpec(shape, lambda i: (0,) * len(shape))
    return pl.pallas_call(
        _final_body,
        grid=(_GRID,),
        in_specs=[
            pl.BlockSpec((2, _BLK, H_FEAT), lambda i: (0, i, 0)),
            pl.BlockSpec((2, _BLK, TILES), lambda i: (0, i, 0)),
            full((1, H_FEAT)),
            full((H_FEAT, N_OUT)),
            full((1, N_OUT)),
        ],
        out_specs=pl.BlockSpec((_BLK, N_OUT), lambda i: (i, 0)),
        out_shape=jax.ShapeDtypeStruct((N_FLOW, N_OUT), jnp.float32),
    )(acc, dens, b12, w_lin, b_lin)


def kernel(x_host, x_flow, ei_src_of, ei_dst_of, ei_rev_src_of, ei_rev_dst_of,
           W_r1, att_s_r1, att_d_r1, b_r1, W_r2, att_s_r2, att_d_r2, b_r2,
           W_r3, att_s_r3, att_d_r3, b_r3, W_r4, att_s_r4, att_d_r4, b_r4,
           W_lin, b_lin):
    h_tbl, a_tbl = _prep(
        x_host, x_flow, W_r1, W_r2,
        att_s_r1.reshape(1, H_FEAT), att_d_r1.reshape(1, H_FEAT),
        att_s_r2.reshape(1, H_FEAT), att_d_r2.reshape(1, H_FEAT))

    h_tbl = h_tbl.reshape(2 * N_HOST, H_FEAT)
    a_flat = a_tbl.T.reshape(4 * N_HOST)
    src_all = jnp.stack([ei_src_of[0], ei_dst_of[0] + N_HOST])
    dst_all = jnp.stack([ei_src_of[1], ei_dst_of[1]])
    src_all = src_all.astype(jnp.int32).reshape(2 * TILES, BLOCKS, NB, CW)
    dst_all = dst_all.astype(jnp.int32).reshape(2 * TILES, BLOCKS, NB, CW)
    zeros = jnp.zeros((ROWS_PT, H_FEAT), jnp.float32)
    zeros_d = jnp.zeros((N_FLOW,), jnp.float32)

    ex_flat, dens = _sc_coef(a_flat, src_all, dst_all, zeros_d)
    acc = _sc_rows(h_tbl, ex_flat, src_all, dst_all, zeros)
    acc = acc.reshape(2, NPAD, H_FEAT)[:, :N_FLOW, :]
    dens = dens.reshape(2, TILES, N_FLOW)
    dens = dens.transpose(0, 2, 1)

    b12 = (b_r1 + b_r2).reshape(1, H_FEAT)
    return _final(acc, dens, b12, W_lin, b_lin.reshape(1, N_OUT))
